# baseline (speedup 1.0000x reference)
import functools

import jax
import jax.numpy as jnp
from jax import lax
from jax.experimental import pallas as pl
from jax.experimental.pallas import tpu as pltpu
from jax.experimental.pallas import tpu_sc as plsc

_D = 128
_CH = 128
_BLK = 4096
_INV_LN2 = 1.4426950408889634


def _gather3(user2d, pos2d, neg2d, user_table, item_table, batch):
    info = plsc.get_sparse_core_info()
    nw = info.num_cores * info.num_subcores
    rows_per_w = batch // nw
    cpg = rows_per_w // _CH
    ntask = 3 * cpg
    mesh = plsc.VectorSubcoreMesh(core_axis_name="c", subcore_axis_name="s")

    @functools.partial(
        pl.kernel,
        mesh=mesh,
        out_type=jax.ShapeDtypeStruct((3, batch, _D), jnp.float32),
        scratch_types=[
            pltpu.VMEM((ntask, _CH), jnp.int32),
            pltpu.VMEM((6, _CH, _D), jnp.float32),
        ] + [pltpu.SemaphoreType.DMA] * 13,
    )
    def k(user_h, pos_h, neg_h, ut_h, it_h, out_h,
          idx_v, rows_v, *sems):
        wid = lax.axis_index("s") * info.num_cores + lax.axis_index("c")
        rbase = wid * rows_per_w
        irow0 = wid * cpg
        gsem = sems[0:6]
        ssem = sems[6:12]
        isem = sems[12]
        ic = [pltpu.async_copy(src.at[pl.ds(irow0, cpg)],
                               idx_v.at[pl.ds(off * cpg, cpg)], isem)
              for off, src in enumerate((user_h, pos_h, neg_h))]

        tasks = []
        for j, tbl in enumerate((ut_h, it_h, it_h)):
            for c in range(cpg):
                tasks.append((j, j * cpg + c, tbl, rbase + c * _CH))

        scat = [None] * 6
        q = []

        def drain_one():
            pg, pb, pj, pobase = q.pop(0)
            pg.wait()
            scat[pb] = pltpu.async_copy(
                rows_v.at[pb], out_h.at[pj, pl.ds(pobase, _CH)], ssem[pb])

        for t, (tj, ti, tbl, obase) in enumerate(tasks):
            if ic[tj] is not None:
                ic[tj].wait()
                ic[tj] = None
            b = t % 6
            if scat[b] is not None:
                scat[b].wait()
                scat[b] = None
            q.append((pltpu.async_copy(tbl.at[idx_v.at[ti]], rows_v.at[b],
                                       gsem[b]), b, tj, obase))
            if len(q) >= 4:
                drain_one()
        while q:
            drain_one()
        for sc in scat:
            if sc is not None:
                sc.wait()

    return k(user2d, pos2d, neg2d, user_table, item_table)


_RT = (((1,), (1,)), ((), ()))


def _mlp_body(x_ref, w0u_ref, w0i_ref, b0_ref,
              w1_ref, b1_ref, w2_ref, b2_ref, wp_ref, bp_ref,
              pp_ref, pn_ref, loss_ref):
    eu = x_ref[0]
    ep = x_ref[1]
    en = x_ref[2]
    aT = lax.dot_general(w0u_ref[...], eu, _RT,
                         preferred_element_type=jnp.float32)
    b0 = b0_ref[...]
    h0p = jnp.maximum(
        aT + lax.dot_general(w0i_ref[...], ep, _RT,
                             preferred_element_type=jnp.float32) + b0, 0.0)
    h0n = jnp.maximum(
        aT + lax.dot_general(w0i_ref[...], en, _RT,
                             preferred_element_type=jnp.float32) + b0, 0.0)
    w1 = w1_ref[...]
    b1 = b1_ref[...]
    h1p = jnp.maximum(
        jnp.dot(w1, h0p, preferred_element_type=jnp.float32) + b1, 0.0)
    h1n = jnp.maximum(
        jnp.dot(w1, h0n, preferred_element_type=jnp.float32) + b1, 0.0)
    w2 = w2_ref[...]
    b2 = b2_ref[...]
    h2p = jnp.maximum(
        jnp.dot(w2, h1p, preferred_element_type=jnp.float32) + b2, 0.0)
    h2n = jnp.maximum(
        jnp.dot(w2, h1n, preferred_element_type=jnp.float32) + b2, 0.0)
    wp = wp_ref[...]
    bp = bp_ref[0, 0]
    pp = jnp.sum(h2p * wp, axis=0, keepdims=True) + bp
    pn = jnp.sum(h2n * wp, axis=0, keepdims=True) + bp
    pp_ref[...] = pp[None]
    pn_ref[...] = pn[None]
    d = pp - pn
    l2 = -(jnp.maximum(-d, 0.0)
           + jnp.log(1.0 + jnp.exp(-jnp.abs(d)))) * _INV_LN2
    blk_loss = jnp.full((1, _D), -jnp.sum(l2), jnp.float32)

    @pl.when(pl.program_id(0) == 0)
    def _init():
        loss_ref[...] = jnp.zeros((1, _D), jnp.float32)

    loss_ref[...] += blk_loss


def _mlp(emb, w0u, w0i, b0r, w1p, b1p, w2p, b2p, wpp, bp11, batch):
    n_blk = batch // _BLK
    row_spec = pl.BlockSpec((3, _BLK, _D), lambda i: (0, i, 0))
    w_spec = pl.BlockSpec((_D, _D), lambda i: (0, 0))
    c_spec = pl.BlockSpec((_D, 1), lambda i: (0, 0))
    return pl.pallas_call(
        _mlp_body,
        grid=(n_blk,),
        in_specs=[row_spec,
                  w_spec, w_spec, c_spec,
                  w_spec, c_spec,
                  w_spec, c_spec,
                  c_spec,
                  pl.BlockSpec(memory_space=pltpu.SMEM)],
        out_specs=[pl.BlockSpec((1, 1, _BLK), lambda i: (i, 0, 0)),
                   pl.BlockSpec((1, 1, _BLK), lambda i: (i, 0, 0)),
                   pl.BlockSpec((1, _D), lambda i: (0, 0))],
        out_shape=[jax.ShapeDtypeStruct((n_blk, 1, _BLK), jnp.float32),
                   jax.ShapeDtypeStruct((n_blk, 1, _BLK), jnp.float32),
                   jax.ShapeDtypeStruct((1, _D), jnp.float32)],
    )(emb, w0u, w0i, b0r, w1p, b1p, w2p, b2p, wpp, bp11)


def kernel(user, pos_item, neg_item, user_table, item_table,
           W0, b0, W1, b1, W2, b2, Wp, bp):
    batch = user.shape[0]
    user2d = user.astype(jnp.int32).reshape(batch // _CH, _CH)
    pos2d = pos_item.astype(jnp.int32).reshape(batch // _CH, _CH)
    neg2d = neg_item.astype(jnp.int32).reshape(batch // _CH, _CH)

    w0u = W0[:_D].T
    w0i = W0[_D:].T
    b0r = b0.reshape(_D, 1)
    w1p = jnp.zeros((_D, _D), jnp.float32).at[:64, :].set(W1.T)
    b1p = jnp.zeros((_D, 1), jnp.float32).at[:64, 0].set(b1)
    w2p = jnp.zeros((_D, _D), jnp.float32).at[:32, :64].set(W2.T)
    b2p = jnp.zeros((_D, 1), jnp.float32).at[:32, 0].set(b2)
    wpp = jnp.zeros((_D, 1), jnp.float32).at[:32, 0].set(Wp[:, 0])
    bp11 = bp.reshape(1, 1)

    emb = _gather3(user2d, pos2d, neg2d, user_table, item_table, batch)
    pp, pn, loss = _mlp(emb, w0u, w0i, b0r, w1p, b1p, w2p, b2p,
                        wpp, bp11, batch)
    return pp.reshape(batch), pn.reshape(batch), loss[0, 0].reshape(())


# speedup vs baseline: 1.0142x; 1.0142x over previous
import functools

import jax
import jax.numpy as jnp
from jax import lax
from jax.experimental import pallas as pl
from jax.experimental.pallas import tpu as pltpu
from jax.experimental.pallas import tpu_sc as plsc

_D = 128
_CH = 128
_BLK = 4096
_INV_LN2 = 1.4426950408889634


def _gather3(user2d, pos2d, neg2d, user_table, item_table, batch):
    info = plsc.get_sparse_core_info()
    nw = info.num_cores * info.num_subcores
    rows_per_w = batch // nw
    cpg = rows_per_w // _CH
    ntask = 3 * cpg
    mesh = plsc.VectorSubcoreMesh(core_axis_name="c", subcore_axis_name="s")

    @functools.partial(
        pl.kernel,
        mesh=mesh,
        out_type=jax.ShapeDtypeStruct((3, batch, _D), jnp.float32),
        scratch_types=[
            pltpu.VMEM((ntask, _CH), jnp.int32),
            pltpu.VMEM((7, _CH, _D), jnp.float32),
        ] + [pltpu.SemaphoreType.DMA] * 15,
    )
    def k(user_h, pos_h, neg_h, ut_h, it_h, out_h,
          idx_v, rows_v, *sems):
        wid = lax.axis_index("s") * info.num_cores + lax.axis_index("c")
        rbase = wid * rows_per_w
        irow0 = wid * cpg
        gsem = sems[0:7]
        ssem = sems[7:14]
        isem = sems[14]
        ic = [pltpu.async_copy(src.at[pl.ds(irow0, cpg)],
                               idx_v.at[pl.ds(off * cpg, cpg)], isem)
              for off, src in enumerate((user_h, pos_h, neg_h))]

        tasks = []
        for j, tbl in enumerate((ut_h, it_h, it_h)):
            for c in range(cpg):
                tasks.append((j, j * cpg + c, tbl, rbase + c * _CH))

        scat = [None] * 7
        q = []

        def drain_one():
            pg, pb, pj, pobase = q.pop(0)
            pg.wait()
            scat[pb] = pltpu.async_copy(
                rows_v.at[pb], out_h.at[pj, pl.ds(pobase, _CH)], ssem[pb])

        for t, (tj, ti, tbl, obase) in enumerate(tasks):
            if ic[tj] is not None:
                ic[tj].wait()
                ic[tj] = None
            b = t % 7
            if scat[b] is not None:
                scat[b].wait()
                scat[b] = None
            q.append((pltpu.async_copy(tbl.at[idx_v.at[ti]], rows_v.at[b],
                                       gsem[b]), b, tj, obase))
            if len(q) >= 5:
                drain_one()
        while q:
            drain_one()
        for sc in scat:
            if sc is not None:
                sc.wait()

    return k(user2d, pos2d, neg2d, user_table, item_table)


_RT = (((1,), (1,)), ((), ()))


def _mlp_body(x_ref, w0u_ref, w0i_ref, b0_ref,
              w1_ref, b1_ref, w2_ref, b2_ref, wp_ref, bp_ref,
              pp_ref, pn_ref, loss_ref):
    eu = x_ref[0]
    ep = x_ref[1]
    en = x_ref[2]
    aT = lax.dot_general(w0u_ref[...], eu, _RT,
                         preferred_element_type=jnp.float32)
    b0 = b0_ref[...]
    h0p = jnp.maximum(
        aT + lax.dot_general(w0i_ref[...], ep, _RT,
                             preferred_element_type=jnp.float32) + b0, 0.0)
    h0n = jnp.maximum(
        aT + lax.dot_general(w0i_ref[...], en, _RT,
                             preferred_element_type=jnp.float32) + b0, 0.0)
    w1 = w1_ref[...]
    b1 = b1_ref[...]
    h1p = jnp.maximum(
        jnp.dot(w1, h0p, preferred_element_type=jnp.float32) + b1, 0.0)
    h1n = jnp.maximum(
        jnp.dot(w1, h0n, preferred_element_type=jnp.float32) + b1, 0.0)
    w2 = w2_ref[...]
    b2 = b2_ref[...]
    h2p = jnp.maximum(
        jnp.dot(w2, h1p, preferred_element_type=jnp.float32) + b2, 0.0)
    h2n = jnp.maximum(
        jnp.dot(w2, h1n, preferred_element_type=jnp.float32) + b2, 0.0)
    wp = wp_ref[...]
    bp = bp_ref[0, 0]
    pp = jnp.sum(h2p * wp, axis=0, keepdims=True) + bp
    pn = jnp.sum(h2n * wp, axis=0, keepdims=True) + bp
    pp_ref[...] = pp[None]
    pn_ref[...] = pn[None]
    d = pp - pn
    l2 = -(jnp.maximum(-d, 0.0)
           + jnp.log(1.0 + jnp.exp(-jnp.abs(d)))) * _INV_LN2
    blk_loss = jnp.full((1, _D), -jnp.sum(l2), jnp.float32)

    @pl.when(pl.program_id(0) == 0)
    def _init():
        loss_ref[...] = jnp.zeros((1, _D), jnp.float32)

    loss_ref[...] += blk_loss


def _mlp(emb, w0u, w0i, b0r, w1p, b1p, w2p, b2p, wpp, bp11, batch):
    n_blk = batch // _BLK
    row_spec = pl.BlockSpec((3, _BLK, _D), lambda i: (0, i, 0))
    w_spec = pl.BlockSpec((_D, _D), lambda i: (0, 0))
    c_spec = pl.BlockSpec((_D, 1), lambda i: (0, 0))
    return pl.pallas_call(
        _mlp_body,
        grid=(n_blk,),
        in_specs=[row_spec,
                  w_spec, w_spec, c_spec,
                  w_spec, c_spec,
                  w_spec, c_spec,
                  c_spec,
                  pl.BlockSpec(memory_space=pltpu.SMEM)],
        out_specs=[pl.BlockSpec((1, 1, _BLK), lambda i: (i, 0, 0)),
                   pl.BlockSpec((1, 1, _BLK), lambda i: (i, 0, 0)),
                   pl.BlockSpec((1, _D), lambda i: (0, 0))],
        out_shape=[jax.ShapeDtypeStruct((n_blk, 1, _BLK), jnp.float32),
                   jax.ShapeDtypeStruct((n_blk, 1, _BLK), jnp.float32),
                   jax.ShapeDtypeStruct((1, _D), jnp.float32)],
    )(emb, w0u, w0i, b0r, w1p, b1p, w2p, b2p, wpp, bp11)


def kernel(user, pos_item, neg_item, user_table, item_table,
           W0, b0, W1, b1, W2, b2, Wp, bp):
    batch = user.shape[0]
    user2d = user.astype(jnp.int32).reshape(batch // _CH, _CH)
    pos2d = pos_item.astype(jnp.int32).reshape(batch // _CH, _CH)
    neg2d = neg_item.astype(jnp.int32).reshape(batch // _CH, _CH)

    w0u = W0[:_D].T
    w0i = W0[_D:].T
    b0r = b0.reshape(_D, 1)
    w1p = jnp.zeros((_D, _D), jnp.float32).at[:64, :].set(W1.T)
    b1p = jnp.zeros((_D, 1), jnp.float32).at[:64, 0].set(b1)
    w2p = jnp.zeros((_D, _D), jnp.float32).at[:32, :64].set(W2.T)
    b2p = jnp.zeros((_D, 1), jnp.float32).at[:32, 0].set(b2)
    wpp = jnp.zeros((_D, 1), jnp.float32).at[:32, 0].set(Wp[:, 0])
    bp11 = bp.reshape(1, 1)

    emb = _gather3(user2d, pos2d, neg2d, user_table, item_table, batch)
    pp, pn, loss = _mlp(emb, w0u, w0i, b0r, w1p, b1p, w2p, b2p,
                        wpp, bp11, batch)
    return pp.reshape(batch), pn.reshape(batch), loss[0, 0].reshape(())
